# baseline (speedup 1.0000x reference)
import functools

import jax
import jax.numpy as jnp
from jax import lax
from jax.experimental import pallas as pl
from jax.experimental.pallas import tpu as pltpu
from jax.experimental.pallas import tpu_sc as plsc

NN = 10000
DD = 128
HH = 128
OO = 64
EE = 320000

NCORES = 2
NSUB = 16
NW = NCORES * NSUB

P = 10240
STRIPE = P // NSUB
TPW = 10240
EP = NW * TPW
CH = 128
NCH = TPW // CH

RB = 1000
GRID = NN // RB

NCHC = (EP // CH) // NSUB
PH = 40

_mesh = plsc.VectorSubcoreMesh(core_axis_name="c", subcore_axis_name="s")
_mesh1 = plsc.VectorSubcoreMesh(core_axis_name="c", subcore_axis_name="s",
                                num_cores=1)


@functools.partial(
    pl.kernel,
    out_type=jax.ShapeDtypeStruct((NCORES, P), jnp.float32),
    mesh=_mesh,
    scratch_types=[
        pltpu.VMEM((TPW,), jnp.int32),
        pltpu.VMEM((P,), jnp.float32),
        pltpu.VMEM((STRIPE,), jnp.float32),
        pltpu.VMEM((STRIPE,), jnp.float32),
        pltpu.VMEM_SHARED((NSUB, P), jnp.float32),
    ],
    compiler_params=pltpu.CompilerParams(needs_layout_passes=False),
)
def _sc_hist(dst_hbm, zeros1_hbm, out_hbm, dstv, hist, accl, tmp, shist):
    c = lax.axis_index("c")
    s = lax.axis_index("s")
    wid = c * NSUB + s
    pltpu.sync_copy(zeros1_hbm, hist)
    pltpu.sync_copy(dst_hbm.at[pl.ds(wid * TPW, TPW)], dstv)
    ones16 = jnp.ones((16,), jnp.float32)

    def chunk_body(k, carry):
        for j in range(4):
            idx = dstv[pl.ds(k * 64 + j * 16, 16)]
            plsc.addupdate_scatter(hist, [idx], ones16)
        return carry

    lax.fori_loop(0, TPW // 64, chunk_body, 0)
    pltpu.sync_copy(hist, shist.at[s])
    plsc.subcore_barrier()

    pltpu.sync_copy(zeros1_hbm.at[pl.ds(0, STRIPE)], accl)

    def red_body(t, carry):
        pltpu.sync_copy(shist.at[t, pl.ds(s * STRIPE, STRIPE)], tmp)
        for i in range(STRIPE // 16):
            sl = pl.ds(i * 16, 16)
            accl[sl] = accl[sl] + tmp[sl]
        return carry

    lax.fori_loop(0, NSUB, red_body, 0)
    pltpu.sync_copy(accl, out_hbm.at[c, pl.ds(s * STRIPE, STRIPE)])


@functools.partial(
    pl.kernel,
    out_type=jax.ShapeDtypeStruct((P, DD), jnp.float32),
    mesh=_mesh1,
    scratch_types=[
        pltpu.VMEM((PH, CH), jnp.int32),
        pltpu.VMEM((PH, CH), jnp.int32),
        pltpu.VMEM((CH, DD), jnp.float32),
        pltpu.VMEM((CH, DD), jnp.float32),
        pltpu.VMEM_SHARED((P, DD), jnp.float32),
        pltpu.SemaphoreType.DMA,
        pltpu.SemaphoreType.DMA,
    ],
)
def _sc_conv(y_hbm, src_hbm, dst_hbm, zeros2_hbm, out_hbm,
             srcv, dstv, rows0, rows1, acc, sem0, sem1):
    s = lax.axis_index("s")
    pltpu.sync_copy(zeros2_hbm, acc.at[pl.ds(s * STRIPE, STRIPE)])
    rows = (rows0, rows1)
    sems = (sem0, sem1)
    plsc.subcore_barrier()

    nph = NCHC // PH
    cbase = s * NCHC

    def phase_body(p, carry):
        pbase = cbase + p * PH
        pltpu.sync_copy(src_hbm.at[pl.ds(pbase, PH)], srcv)
        pltpu.sync_copy(dst_hbm.at[pl.ds(pbase, PH)], dstv)
        pltpu.async_copy(y_hbm.at[srcv.at[0]], rows0, sem0)
        pltpu.async_copy(y_hbm.at[srcv.at[1]], rows1, sem1)

        def chunk_body(kk, carry2):
            for b in range(2):
                k = kk * 2 + b
                pltpu.make_async_copy(y_hbm.at[srcv.at[k]], rows[b],
                                      sems[b]).wait()
                pltpu.sync_copy(rows[b], acc.at[dstv.at[k]], add=True)

                @pl.when(k + 2 < PH)
                def _():
                    pltpu.async_copy(y_hbm.at[srcv.at[k + 2]], rows[b],
                                     sems[b])
            return carry2

        lax.fori_loop(0, PH // 2, chunk_body, 0)
        return carry

    lax.fori_loop(0, nph, phase_body, 0)
    plsc.subcore_barrier()
    pltpu.sync_copy(acc.at[pl.ds(s * STRIPE, STRIPE)],
                    out_hbm.at[pl.ds(s * STRIPE, STRIPE)])


_TC_PARAMS = pltpu.CompilerParams(dimension_semantics=("arbitrary",))


def _dinv_body(cnt_ref, dinv_ref):
    dinv_ref[...] = lax.rsqrt(cnt_ref[0:1, :] + cnt_ref[1:2, :] + 1.0)


def _dinv_call(counts):
    return pl.pallas_call(
        _dinv_body,
        out_shape=jax.ShapeDtypeStruct((1, P), jnp.float32),
    )(counts)


def _c1_body(x_ref, rw_ref, rb_ref, w1_ref, dinv_ref, xproj_ref, y1_ref):
    xp = jnp.dot(x_ref[...], rw_ref[...],
                 preferred_element_type=jnp.float32) + rb_ref[...]
    xproj_ref[...] = xp
    y1_ref[...] = jnp.dot(xp, w1_ref[...],
                          preferred_element_type=jnp.float32) * dinv_ref[...]


def _c1_call(x, res_W, res_b, W1, dinv_col):
    return pl.pallas_call(
        _c1_body,
        grid=(GRID,),
        in_specs=[
            pl.BlockSpec((RB, DD), lambda i: (i, 0)),
            pl.BlockSpec((DD, HH), lambda i: (0, 0)),
            pl.BlockSpec((1, HH), lambda i: (0, 0)),
            pl.BlockSpec((HH, HH), lambda i: (0, 0)),
            pl.BlockSpec((RB, 1), lambda i: (i, 0)),
        ],
        out_specs=[
            pl.BlockSpec((RB, HH), lambda i: (i, 0)),
            pl.BlockSpec((RB, HH), lambda i: (i, 0)),
        ],
        out_shape=[
            jax.ShapeDtypeStruct((NN, HH), jnp.float32),
            jax.ShapeDtypeStruct((P, HH), jnp.float32),
        ],
        compiler_params=_TC_PARAMS,
    )(x, res_W, res_b, W1, dinv_col)


def _post_body(p_ref, y_ref, dinv_ref, b_ref, out_ref, st_ref):
    o = dinv_ref[...] * (p_ref[...] + y_ref[...]) + b_ref[...]
    out_ref[...] = o

    @pl.when(pl.program_id(0) == 0)
    def _():
        st_ref[...] = jnp.zeros_like(st_ref)

    s0 = jnp.sum(o, axis=0, keepdims=True)
    s1 = jnp.sum(o * o, axis=0, keepdims=True)
    st_ref[...] += jnp.concatenate([s0, s1], axis=0)


def _post_call(partials, y, dinv_col, b):
    return pl.pallas_call(
        _post_body,
        grid=(GRID,),
        in_specs=[
            pl.BlockSpec((RB, HH), lambda i: (i, 0)),
            pl.BlockSpec((RB, HH), lambda i: (i, 0)),
            pl.BlockSpec((RB, 1), lambda i: (i, 0)),
            pl.BlockSpec((1, HH), lambda i: (0, 0)),
        ],
        out_specs=[
            pl.BlockSpec((RB, HH), lambda i: (i, 0)),
            pl.BlockSpec((2, HH), lambda i: (0, 0)),
        ],
        out_shape=[
            jax.ShapeDtypeStruct((NN, HH), jnp.float32),
            jax.ShapeDtypeStruct((2, HH), jnp.float32),
        ],
        compiler_params=_TC_PARAMS,
    )(partials, y, dinv_col, b)


def _bn_coeffs(st_ref, g_ref, be_ref):
    m = st_ref[0:1, :] * (1.0 / NN)
    v = st_ref[1:2, :] * (1.0 / NN) - m * m
    a = g_ref[...] * lax.rsqrt(v + 1e-5)
    return a, be_ref[...] - m * a


def _c2_body(o1_ref, st_ref, g_ref, be_ref, w2_ref, dinv_ref, y2_ref):
    a, cc = _bn_coeffs(st_ref, g_ref, be_ref)
    h = jnp.maximum(o1_ref[...] * a + cc, 0.0)
    y2_ref[...] = jnp.dot(h, w2_ref[...],
                          preferred_element_type=jnp.float32) * dinv_ref[...]


def _c2_call(out1, st1, g1, be1, W2, dinv_col):
    return pl.pallas_call(
        _c2_body,
        grid=(GRID,),
        in_specs=[
            pl.BlockSpec((RB, HH), lambda i: (i, 0)),
            pl.BlockSpec((2, HH), lambda i: (0, 0)),
            pl.BlockSpec((1, HH), lambda i: (0, 0)),
            pl.BlockSpec((1, HH), lambda i: (0, 0)),
            pl.BlockSpec((HH, HH), lambda i: (0, 0)),
            pl.BlockSpec((RB, 1), lambda i: (i, 0)),
        ],
        out_specs=pl.BlockSpec((RB, HH), lambda i: (i, 0)),
        out_shape=jax.ShapeDtypeStruct((P, HH), jnp.float32),
        compiler_params=_TC_PARAMS,
    )(out1, st1, g1, be1, W2, dinv_col)


def _cls1_body(o2_ref, st_ref, g_ref, be_ref, xp_ref, wc1_ref, bc1_ref,
               t_ref, stt_ref):
    a, cc = _bn_coeffs(st_ref, g_ref, be_ref)
    h = jnp.maximum(o2_ref[...] * a + cc, 0.0) + xp_ref[...]
    t = jnp.dot(h, wc1_ref[...],
                preferred_element_type=jnp.float32) + bc1_ref[...]
    t_ref[...] = t

    @pl.when(pl.program_id(0) == 0)
    def _():
        stt_ref[...] = jnp.zeros_like(stt_ref)

    s0 = jnp.sum(t, axis=0, keepdims=True)
    s1 = jnp.sum(t * t, axis=0, keepdims=True)
    stt_ref[...] += jnp.concatenate([s0, s1], axis=0)


def _cls1_call(out2, st2, g2, be2, xproj, Wc1, bc1):
    return pl.pallas_call(
        _cls1_body,
        grid=(GRID,),
        in_specs=[
            pl.BlockSpec((RB, HH), lambda i: (i, 0)),
            pl.BlockSpec((2, HH), lambda i: (0, 0)),
            pl.BlockSpec((1, HH), lambda i: (0, 0)),
            pl.BlockSpec((1, HH), lambda i: (0, 0)),
            pl.BlockSpec((RB, HH), lambda i: (i, 0)),
            pl.BlockSpec((HH, OO), lambda i: (0, 0)),
            pl.BlockSpec((1, OO), lambda i: (0, 0)),
        ],
        out_specs=[
            pl.BlockSpec((RB, OO), lambda i: (i, 0)),
            pl.BlockSpec((2, OO), lambda i: (0, 0)),
        ],
        out_shape=[
            jax.ShapeDtypeStruct((NN, OO), jnp.float32),
            jax.ShapeDtypeStruct((2, OO), jnp.float32),
        ],
        compiler_params=_TC_PARAMS,
    )(out2, st2, g2, be2, xproj, Wc1, bc1)


def _cls2_body(t_ref, st_ref, g_ref, be_ref, wc2_ref, bc2_ref, out_ref):
    a, cc = _bn_coeffs(st_ref, g_ref, be_ref)
    z = jnp.maximum(t_ref[...] * a + cc, 0.0)
    lg = jnp.dot(z, wc2_ref[...],
                 preferred_element_type=jnp.float32) + bc2_ref[...]
    mx = jnp.max(lg, axis=1, keepdims=True)
    lse = jnp.log(jnp.sum(jnp.exp(lg - mx), axis=1, keepdims=True)) + mx
    out_ref[...] = lg - lse


def _cls2_call(t, stt, gc, bec, Wc2, bc2):
    return pl.pallas_call(
        _cls2_body,
        grid=(GRID,),
        in_specs=[
            pl.BlockSpec((RB, OO), lambda i: (i, 0)),
            pl.BlockSpec((2, OO), lambda i: (0, 0)),
            pl.BlockSpec((1, OO), lambda i: (0, 0)),
            pl.BlockSpec((1, OO), lambda i: (0, 0)),
            pl.BlockSpec((OO, OO), lambda i: (0, 0)),
            pl.BlockSpec((1, OO), lambda i: (0, 0)),
        ],
        out_specs=pl.BlockSpec((RB, OO), lambda i: (i, 0)),
        out_shape=jax.ShapeDtypeStruct((NN, OO), jnp.float32),
        compiler_params=_TC_PARAMS,
    )(t, stt, gc, bec, Wc2, bc2)


def kernel(x, edge_index, res_W, res_b, W1, b1, g1, be1, W2, b2, g2, be2,
           Wc1, bc1, gc, bec, Wc2, bc2):
    pad = jnp.full((EP - EE,), NN, jnp.int32)
    srcp = jnp.concatenate([edge_index[0], pad])
    dstp = jnp.concatenate([edge_index[1], pad])
    src2 = srcp.reshape(EP // CH, CH)
    dst2 = dstp.reshape(EP // CH, CH)
    zeros1 = jnp.zeros((P,), jnp.float32)
    zeros2 = jnp.zeros((STRIPE, DD), jnp.float32)

    counts = _sc_hist(dstp, zeros1)
    dinv_col = _dinv_call(counts).reshape(P, 1)

    xproj, y1 = _c1_call(x, res_W, res_b.reshape(1, HH), W1, dinv_col)
    p1 = _sc_conv(y1, src2, dst2, zeros2)
    out1, st1 = _post_call(p1, y1, dinv_col, b1.reshape(1, HH))

    y2 = _c2_call(out1, st1, g1.reshape(1, HH), be1.reshape(1, HH), W2,
                  dinv_col)
    p2 = _sc_conv(y2, src2, dst2, zeros2)
    out2, st2 = _post_call(p2, y2, dinv_col, b2.reshape(1, HH))

    t, stt = _cls1_call(out2, st2, g2.reshape(1, HH), be2.reshape(1, HH),
                        xproj, Wc1, bc1.reshape(1, OO))
    return _cls2_call(t, stt, gc.reshape(1, OO), bec.reshape(1, OO), Wc2,
                      bc2.reshape(1, OO))


# speedup vs baseline: 1.2117x; 1.2117x over previous
import functools

import jax
import jax.numpy as jnp
from jax import lax
from jax.experimental import pallas as pl
from jax.experimental.pallas import tpu as pltpu
from jax.experimental.pallas import tpu_sc as plsc

NN = 10000
DD = 128
HH = 128
OO = 64
EE = 320000

NCORES = 2
NSUB = 16
NW = NCORES * NSUB

P = 10240
STRIPE = P // NSUB
TPW = 10240
EP = NW * TPW
CH = 128
NCH = TPW // CH

RB = 1000
GRID = NN // RB

NC0 = 120
NC1 = 40
PH = 40

_mesh = plsc.VectorSubcoreMesh(core_axis_name="c", subcore_axis_name="s")
_mesh1 = plsc.VectorSubcoreMesh(core_axis_name="c", subcore_axis_name="s",
                                num_cores=1)


@functools.partial(
    pl.kernel,
    out_type=jax.ShapeDtypeStruct((NCORES, P), jnp.float32),
    mesh=_mesh,
    scratch_types=[
        pltpu.VMEM((TPW,), jnp.int32),
        pltpu.VMEM((P,), jnp.float32),
        pltpu.VMEM((STRIPE,), jnp.float32),
        pltpu.VMEM((STRIPE,), jnp.float32),
        pltpu.VMEM_SHARED((NSUB, P), jnp.float32),
    ],
    compiler_params=pltpu.CompilerParams(needs_layout_passes=False),
)
def _sc_hist(dst_hbm, zeros1_hbm, out_hbm, dstv, hist, accl, tmp, shist):
    c = lax.axis_index("c")
    s = lax.axis_index("s")
    wid = c * NSUB + s
    pltpu.sync_copy(zeros1_hbm, hist)
    pltpu.sync_copy(dst_hbm.at[pl.ds(wid * TPW, TPW)], dstv)
    ones16 = jnp.ones((16,), jnp.float32)

    def chunk_body(k, carry):
        for j in range(4):
            idx = dstv[pl.ds(k * 64 + j * 16, 16)]
            plsc.addupdate_scatter(hist, [idx], ones16)
        return carry

    lax.fori_loop(0, TPW // 64, chunk_body, 0)
    pltpu.sync_copy(hist, shist.at[s])
    plsc.subcore_barrier()

    pltpu.sync_copy(zeros1_hbm.at[pl.ds(0, STRIPE)], accl)

    def red_body(t, carry):
        pltpu.sync_copy(shist.at[t, pl.ds(s * STRIPE, STRIPE)], tmp)
        for i in range(STRIPE // 16):
            sl = pl.ds(i * 16, 16)
            accl[sl] = accl[sl] + tmp[sl]
        return carry

    lax.fori_loop(0, NSUB, red_body, 0)
    pltpu.sync_copy(accl, out_hbm.at[c, pl.ds(s * STRIPE, STRIPE)])


@functools.partial(
    pl.kernel,
    out_type=jax.ShapeDtypeStruct((NCORES, P, DD), jnp.float32),
    mesh=_mesh,
    scratch_types=[
        pltpu.VMEM((PH, CH), jnp.int32),
        pltpu.VMEM((PH, CH), jnp.int32),
        pltpu.VMEM((CH, DD), jnp.float32),
        pltpu.VMEM((CH, DD), jnp.float32),
        pltpu.VMEM_SHARED((P, DD), jnp.float32),
        pltpu.SemaphoreType.DMA,
        pltpu.SemaphoreType.DMA,
    ],
)
def _sc_conv(y_hbm, src_hbm, dst_hbm, zeros2_hbm, out_hbm,
             srcv, dstv, rows0, rows1, acc, sem0, sem1):
    c = lax.axis_index("c")
    s = lax.axis_index("s")
    pltpu.sync_copy(zeros2_hbm, acc.at[pl.ds(s * STRIPE, STRIPE)])
    rows = (rows0, rows1)
    sems = (sem0, sem1)
    plsc.subcore_barrier()

    nph = jnp.where(c == 0, NC0 // PH, NC1 // PH)
    cbase = jnp.where(c == 0, s * NC0, NSUB * NC0 + s * NC1)

    def phase_body(p, carry):
        pbase = cbase + p * PH
        pltpu.sync_copy(src_hbm.at[pl.ds(pbase, PH)], srcv)
        pltpu.sync_copy(dst_hbm.at[pl.ds(pbase, PH)], dstv)
        pltpu.async_copy(y_hbm.at[srcv.at[0]], rows0, sem0)
        pltpu.async_copy(y_hbm.at[srcv.at[1]], rows1, sem1)

        def chunk_body(kk, carry2):
            for b in range(2):
                k = kk * 2 + b
                pltpu.make_async_copy(y_hbm.at[srcv.at[k]], rows[b],
                                      sems[b]).wait()
                pltpu.sync_copy(rows[b], acc.at[dstv.at[k]], add=True)

                @pl.when(k + 2 < PH)
                def _():
                    pltpu.async_copy(y_hbm.at[srcv.at[k + 2]], rows[b],
                                     sems[b])
            return carry2

        lax.fori_loop(0, PH // 2, chunk_body, 0)
        return carry

    lax.fori_loop(0, nph, phase_body, 0)
    plsc.subcore_barrier()
    pltpu.sync_copy(acc.at[pl.ds(s * STRIPE, STRIPE)],
                    out_hbm.at[c, pl.ds(s * STRIPE, STRIPE)])


_TC_PARAMS = pltpu.CompilerParams(dimension_semantics=("arbitrary",))


def _dinv_body(cnt_ref, dinv_ref):
    dinv_ref[...] = lax.rsqrt(cnt_ref[0:1, :] + cnt_ref[1:2, :] + 1.0)


def _dinv_call(counts):
    return pl.pallas_call(
        _dinv_body,
        out_shape=jax.ShapeDtypeStruct((1, P), jnp.float32),
    )(counts)


def _c1_body(x_ref, rw_ref, rb_ref, w1_ref, dinv_ref, xproj_ref, y1_ref):
    xp = jnp.dot(x_ref[...], rw_ref[...],
                 preferred_element_type=jnp.float32) + rb_ref[...]
    xproj_ref[...] = xp
    y1_ref[...] = jnp.dot(xp, w1_ref[...],
                          preferred_element_type=jnp.float32) * dinv_ref[...]


def _c1_call(x, res_W, res_b, W1, dinv_col):
    return pl.pallas_call(
        _c1_body,
        grid=(GRID,),
        in_specs=[
            pl.BlockSpec((RB, DD), lambda i: (i, 0)),
            pl.BlockSpec((DD, HH), lambda i: (0, 0)),
            pl.BlockSpec((1, HH), lambda i: (0, 0)),
            pl.BlockSpec((HH, HH), lambda i: (0, 0)),
            pl.BlockSpec((RB, 1), lambda i: (i, 0)),
        ],
        out_specs=[
            pl.BlockSpec((RB, HH), lambda i: (i, 0)),
            pl.BlockSpec((RB, HH), lambda i: (i, 0)),
        ],
        out_shape=[
            jax.ShapeDtypeStruct((NN, HH), jnp.float32),
            jax.ShapeDtypeStruct((P, HH), jnp.float32),
        ],
        compiler_params=_TC_PARAMS,
    )(x, res_W, res_b, W1, dinv_col)


def _post_body(p_ref, y_ref, dinv_ref, b_ref, out_ref, st_ref):
    o = dinv_ref[...] * (p_ref[0] + p_ref[1] + y_ref[...]) + b_ref[...]
    out_ref[...] = o

    @pl.when(pl.program_id(0) == 0)
    def _():
        st_ref[...] = jnp.zeros_like(st_ref)

    s0 = jnp.sum(o, axis=0, keepdims=True)
    s1 = jnp.sum(o * o, axis=0, keepdims=True)
    st_ref[...] += jnp.concatenate([s0, s1], axis=0)


def _post_call(partials, y, dinv_col, b):
    return pl.pallas_call(
        _post_body,
        grid=(GRID,),
        in_specs=[
            pl.BlockSpec((NCORES, RB, HH), lambda i: (0, i, 0)),
            pl.BlockSpec((RB, HH), lambda i: (i, 0)),
            pl.BlockSpec((RB, 1), lambda i: (i, 0)),
            pl.BlockSpec((1, HH), lambda i: (0, 0)),
        ],
        out_specs=[
            pl.BlockSpec((RB, HH), lambda i: (i, 0)),
            pl.BlockSpec((2, HH), lambda i: (0, 0)),
        ],
        out_shape=[
            jax.ShapeDtypeStruct((NN, HH), jnp.float32),
            jax.ShapeDtypeStruct((2, HH), jnp.float32),
        ],
        compiler_params=_TC_PARAMS,
    )(partials, y, dinv_col, b)


def _bn_coeffs(st_ref, g_ref, be_ref):
    m = st_ref[0:1, :] * (1.0 / NN)
    v = st_ref[1:2, :] * (1.0 / NN) - m * m
    a = g_ref[...] * lax.rsqrt(v + 1e-5)
    return a, be_ref[...] - m * a


def _c2_body(o1_ref, st_ref, g_ref, be_ref, w2_ref, dinv_ref, y2_ref):
    a, cc = _bn_coeffs(st_ref, g_ref, be_ref)
    h = jnp.maximum(o1_ref[...] * a + cc, 0.0)
    y2_ref[...] = jnp.dot(h, w2_ref[...],
                          preferred_element_type=jnp.float32) * dinv_ref[...]


def _c2_call(out1, st1, g1, be1, W2, dinv_col):
    return pl.pallas_call(
        _c2_body,
        grid=(GRID,),
        in_specs=[
            pl.BlockSpec((RB, HH), lambda i: (i, 0)),
            pl.BlockSpec((2, HH), lambda i: (0, 0)),
            pl.BlockSpec((1, HH), lambda i: (0, 0)),
            pl.BlockSpec((1, HH), lambda i: (0, 0)),
            pl.BlockSpec((HH, HH), lambda i: (0, 0)),
            pl.BlockSpec((RB, 1), lambda i: (i, 0)),
        ],
        out_specs=pl.BlockSpec((RB, HH), lambda i: (i, 0)),
        out_shape=jax.ShapeDtypeStruct((P, HH), jnp.float32),
        compiler_params=_TC_PARAMS,
    )(out1, st1, g1, be1, W2, dinv_col)


def _cls1_body(o2_ref, st_ref, g_ref, be_ref, xp_ref, wc1_ref, bc1_ref,
               t_ref, stt_ref):
    a, cc = _bn_coeffs(st_ref, g_ref, be_ref)
    h = jnp.maximum(o2_ref[...] * a + cc, 0.0) + xp_ref[...]
    t = jnp.dot(h, wc1_ref[...],
                preferred_element_type=jnp.float32) + bc1_ref[...]
    t_ref[...] = t

    @pl.when(pl.program_id(0) == 0)
    def _():
        stt_ref[...] = jnp.zeros_like(stt_ref)

    s0 = jnp.sum(t, axis=0, keepdims=True)
    s1 = jnp.sum(t * t, axis=0, keepdims=True)
    stt_ref[...] += jnp.concatenate([s0, s1], axis=0)


def _cls1_call(out2, st2, g2, be2, xproj, Wc1, bc1):
    return pl.pallas_call(
        _cls1_body,
        grid=(GRID,),
        in_specs=[
            pl.BlockSpec((RB, HH), lambda i: (i, 0)),
            pl.BlockSpec((2, HH), lambda i: (0, 0)),
            pl.BlockSpec((1, HH), lambda i: (0, 0)),
            pl.BlockSpec((1, HH), lambda i: (0, 0)),
            pl.BlockSpec((RB, HH), lambda i: (i, 0)),
            pl.BlockSpec((HH, OO), lambda i: (0, 0)),
            pl.BlockSpec((1, OO), lambda i: (0, 0)),
        ],
        out_specs=[
            pl.BlockSpec((RB, OO), lambda i: (i, 0)),
            pl.BlockSpec((2, OO), lambda i: (0, 0)),
        ],
        out_shape=[
            jax.ShapeDtypeStruct((NN, OO), jnp.float32),
            jax.ShapeDtypeStruct((2, OO), jnp.float32),
        ],
        compiler_params=_TC_PARAMS,
    )(out2, st2, g2, be2, xproj, Wc1, bc1)


def _cls2_body(t_ref, st_ref, g_ref, be_ref, wc2_ref, bc2_ref, out_ref):
    a, cc = _bn_coeffs(st_ref, g_ref, be_ref)
    z = jnp.maximum(t_ref[...] * a + cc, 0.0)
    lg = jnp.dot(z, wc2_ref[...],
                 preferred_element_type=jnp.float32) + bc2_ref[...]
    mx = jnp.max(lg, axis=1, keepdims=True)
    lse = jnp.log(jnp.sum(jnp.exp(lg - mx), axis=1, keepdims=True)) + mx
    out_ref[...] = lg - lse


def _cls2_call(t, stt, gc, bec, Wc2, bc2):
    return pl.pallas_call(
        _cls2_body,
        grid=(GRID,),
        in_specs=[
            pl.BlockSpec((RB, OO), lambda i: (i, 0)),
            pl.BlockSpec((2, OO), lambda i: (0, 0)),
            pl.BlockSpec((1, OO), lambda i: (0, 0)),
            pl.BlockSpec((1, OO), lambda i: (0, 0)),
            pl.BlockSpec((OO, OO), lambda i: (0, 0)),
            pl.BlockSpec((1, OO), lambda i: (0, 0)),
        ],
        out_specs=pl.BlockSpec((RB, OO), lambda i: (i, 0)),
        out_shape=jax.ShapeDtypeStruct((NN, OO), jnp.float32),
        compiler_params=_TC_PARAMS,
    )(t, stt, gc, bec, Wc2, bc2)


def kernel(x, edge_index, res_W, res_b, W1, b1, g1, be1, W2, b2, g2, be2,
           Wc1, bc1, gc, bec, Wc2, bc2):
    pad = jnp.full((EP - EE,), NN, jnp.int32)
    srcp = jnp.concatenate([edge_index[0], pad])
    dstp = jnp.concatenate([edge_index[1], pad])
    src2 = srcp.reshape(EP // CH, CH)
    dst2 = dstp.reshape(EP // CH, CH)
    zeros1 = jnp.zeros((P,), jnp.float32)
    zeros2 = jnp.zeros((STRIPE, DD), jnp.float32)

    counts = _sc_hist(dstp, zeros1)
    dinv_col = _dinv_call(counts).reshape(P, 1)

    xproj, y1 = _c1_call(x, res_W, res_b.reshape(1, HH), W1, dinv_col)
    p1 = _sc_conv(y1, src2, dst2, zeros2)
    out1, st1 = _post_call(p1, y1, dinv_col, b1.reshape(1, HH))

    y2 = _c2_call(out1, st1, g1.reshape(1, HH), be1.reshape(1, HH), W2,
                  dinv_col)
    p2 = _sc_conv(y2, src2, dst2, zeros2)
    out2, st2 = _post_call(p2, y2, dinv_col, b2.reshape(1, HH))

    t, stt = _cls1_call(out2, st2, g2.reshape(1, HH), be2.reshape(1, HH),
                        xproj, Wc1, bc1.reshape(1, OO))
    return _cls2_call(t, stt, gc.reshape(1, OO), bec.reshape(1, OO), Wc2,
                      bc2.reshape(1, OO))
